# baseline (speedup 1.0000x reference)
import functools

import jax
import jax.numpy as jnp
import numpy as np
from jax import lax
from jax.experimental import pallas as pl
from jax.experimental.pallas import tpu as pltpu
from jax.experimental.pallas import tpu_sc as plsc

_EPS = 1e-5
_K = 16
_NT = 256
_SPLIT0 = 100


def _sc_gather_pack(table, idx_flat, n_rows, keep):
    info = plsc.get_sparse_core_info()
    nc, ns = info.num_cores, info.num_subcores
    total_chunks = n_rows // 128
    ch0 = ((total_chunks * _SPLIT0) // (ns * 100) // 8) * 8
    ch1 = total_chunks // ns - ch0
    max_ch = max(ch0, ch1, 2)
    pack = 128 // keep
    rows_pc = 128 // pack
    out_rows = n_rows // pack
    idx2 = idx_flat.reshape(total_chunks, 128)
    mesh = plsc.VectorSubcoreMesh(core_axis_name="c", subcore_axis_name="s")

    @functools.partial(
        pl.kernel,
        mesh=mesh,
        out_type=jax.ShapeDtypeStruct((out_rows, 128), jnp.float32),
        scratch_types=[
            pltpu.VMEM((max_ch, 128), jnp.int32),
            pltpu.VMEM((4, 128, 128), jnp.float32),
            pltpu.VMEM((2, rows_pc, 128), jnp.float32),
            pltpu.SemaphoreType.DMA,
            pltpu.SemaphoreType.DMA,
            pltpu.SemaphoreType.DMA,
            pltpu.SemaphoreType.DMA,
            pltpu.SemaphoreType.DMA,
            pltpu.SemaphoreType.DMA,
        ],
    )
    def gk(table_hbm, idx_hbm, out_hbm, idx_v, gbuf, cpk,
           gs0, gs1, gs2, gs3, ss0, ss1):
        cid = lax.axis_index("c")
        sid = lax.axis_index("s")
        gsems = (gs0, gs1, gs2, gs3)
        ssems = (ss0, ss1)

        def compact(b, p):
            def crow(r, carry):
                for l in range(pack):
                    for c4 in range(keep // 16):
                        v = gbuf[b, pack * r + l, pl.ds(c4 * 16, 16)]
                        cpk[p, r, pl.ds(l * keep + c4 * 16, 16)] = v
                return carry
            lax.fori_loop(0, rows_pc, crow, 0)

        def run(cb, cpw):
            if cpw == 0:
                return
            ro = cb * rows_pc
            pltpu.sync_copy(idx_hbm.at[pl.ds(cb, cpw)],
                            idx_v.at[pl.ds(0, cpw)])
            for b in range(4):
                pltpu.async_copy(table_hbm.at[idx_v.at[b]], gbuf.at[b],
                                 gsems[b])

            def step(s, carry):
                for b in range(4):
                    c = 4 * s + b
                    p = b % 2
                    pltpu.make_async_copy(
                        table_hbm.at[idx_v.at[0]], gbuf.at[b], gsems[b]).wait()

                    @pl.when(c >= 2)
                    def _drain_store():
                        pltpu.make_async_copy(
                            cpk.at[p], out_hbm.at[pl.ds(ro, rows_pc)],
                            ssems[p]).wait()

                    compact(b, p)
                    pltpu.async_copy(
                        cpk.at[p],
                        out_hbm.at[pl.ds(ro + c * rows_pc, rows_pc)],
                        ssems[p])

                    @pl.when(c + 4 < cpw)
                    def _next():
                        pltpu.async_copy(
                            table_hbm.at[idx_v.at[c + 4]], gbuf.at[b],
                            gsems[b])
                return carry

            lax.fori_loop(0, cpw // 4, step, 0)
            for p in range(2):
                pltpu.make_async_copy(
                    cpk.at[p], out_hbm.at[pl.ds(ro, rows_pc)], ssems[p]).wait()

        @pl.when(cid == 0)
        def _c0():
            run(sid * ch0, ch0)

        @pl.when(cid == 1)
        def _c1():
            run(ns * ch0 + sid * ch1, ch1)

    return gk(table, idx2)


def _tc1_body(feat, xyzp, nbp_ref, t8, sd, m0, sha, shb, wfcbd, rmat,
              w1p, b1r, wattp, battp, wae, waf, bg1r, wbe, wbf,
              a_ref, bv_ref, enc_ref):
    nt = _NT
    xyz16 = xyzp[...]
    nbp = nbp_ref[...]
    cen1 = jnp.dot(xyz16, t8[...])
    cenp = jnp.broadcast_to(cen1[:, None, :], (nt, 2, 128)).reshape(nt * 2, 128)
    relp = cenp - nbp
    ssq = jnp.dot(relp * relp, sd[...])
    distp = jnp.sqrt(ssq + 1e-12)
    fx = distp * m0[...] + jnp.dot(cenp, sha[...]) + jnp.dot(nbp, shb[...])
    logits = jnp.dot(fx, wfcbd[...])
    e = jnp.exp(logits)
    s2 = jnp.sum(e.reshape(nt, 2, 128), axis=1)
    den = jnp.dot(s2, rmat[...])
    attn = e / jnp.broadcast_to(den[:, None, :], (nt, 2, 128)).reshape(nt * 2, 128)
    w = fx * attn
    fagg_rep = jnp.dot(jnp.sum(w.reshape(nt, 2, 128), axis=1), rmat[...])
    fagg = fagg_rep[:, :16]
    enc = jnp.maximum(jnp.dot(fagg, wattp[...]) + battp[...], 0.0)
    f_pc = jnp.maximum(
        lax.dot_general(feat[...], w1p[...], (((0,), (0,)), ((), ()))) + b1r[...],
        0.0)
    a_ref[...] = jnp.dot(enc, wae[...]) + jnp.dot(f_pc, waf[...]) + bg1r[...]
    bv_ref[...] = jnp.dot(enc, wbe[...]) + jnp.dot(f_pc, wbf[...])
    enc_ref[...] = enc


def _tc2_body(g_ref, a_ref, feat_ref, dup, wgpk, bg2r, w2, ws, bsum, out_ref):
    nt = _NT
    gp = g_ref[...]
    a = a_ref[...]
    aw = jnp.dot(a, dup[...])
    ap = jnp.broadcast_to(aw[:, None, :], (nt, 8, 128)).reshape(nt * 8, 128)
    h = jnp.maximum(gp + ap, 0.0)
    h2 = lax.dot_general(h, wgpk[...], (((1,), (0,)), ((), ())))
    m8 = jnp.max(h2.reshape(nt, 8, 256), axis=1)
    mx = jnp.maximum(jnp.maximum(m8[:, :128], m8[:, 128:]) + bg2r[...], 0.0)
    y = (lax.dot_general(mx, w2[...], (((1,), (1,)), ((), ())))
         + lax.dot_general(feat_ref[...], ws[...], (((0,), (1,)), ((), ())))
         + bsum[...])
    out_ref[...] = jnp.maximum(y, 0.2 * y)


def kernel(feature, xyz, neigh_idx, encode_list, W1, g1, b1, Wfc, Watt, gatt,
           batt, Wg1, gg1, bg1, Wg2, gg2, bg2, W2, g2, b2, Ws, gs, bs):
    del encode_list
    B, d_in, N, _ = feature.shape
    k = neigh_idx.shape[-1]
    npad = ((N + _NT - 1) // _NT) * _NT
    ep = npad * k
    s = 1.0 / np.sqrt(1.0 + _EPS)

    feat = feature[0, :, :, 0]
    featp = jnp.pad(feat, ((0, 0), (0, npad - N)))
    xyzt = jnp.pad(xyz[0], ((0, npad - N), (0, 125)))
    xyzp = jnp.pad(xyz[0], ((0, npad - N), (0, 13)))
    idx_flat = jnp.pad(neigh_idx[0], ((0, npad - N), (0, 0))).astype(
        jnp.int32).reshape(ep)

    w1p = (W1 * (g1 * s)[:, None]).T
    b1r = b1[None, :]
    m0_np = np.zeros((1, 128), np.float32)
    m0_np[0, 0::16] = 1.0
    rmat_np = np.zeros((128, 128), np.float32)
    t8_np = np.zeros((16, 128), np.float32)
    sd_np = np.zeros((128, 128), np.float32)
    sh1_np = np.zeros((128, 128), np.float32)
    sh4_np = np.zeros((128, 128), np.float32)
    sh7_np = np.zeros((128, 128), np.float32)
    for kb in range(8):
        o = 16 * kb
        for c in range(16):
            t8_np[c, o + c] = 1.0
        for c in range(3):
            sd_np[o + c, o] = 1.0
            sh1_np[o + c, o + 1 + c] = 1.0
            sh4_np[o + c, o + 4 + c] = 1.0
            sh7_np[o + c, o + 7 + c] = 1.0
    for i in range(128):
        for j in range(i % 16, 128, 16):
            rmat_np[i, j] = 1.0
    m0c = jnp.asarray(m0_np)
    rmat = jnp.asarray(rmat_np)
    t8 = jnp.asarray(t8_np)
    sd = jnp.asarray(sd_np)
    sha = jnp.asarray(sh1_np + sh4_np)
    shb = jnp.asarray(sh7_np - sh1_np)
    wfcp = jnp.zeros((16, 16), jnp.float32).at[:10, :10].set(Wfc.T)
    wfcbd = jax.scipy.linalg.block_diag(*([wfcp] * 8))
    wattf = Watt * (gatt * s)[:, None]
    wattp = jnp.zeros((16, 16), jnp.float32).at[:10, :10].set(wattf.T)
    battp = jnp.zeros((1, 16), jnp.float32).at[0, :10].set(batt)
    wg1f = Wg1 * (gg1 * s)[:, None]
    wa = wg1f[:, :74] - wg1f[:, 74:]
    wb = wg1f[:, 74:]
    wae = jnp.zeros((16, 64), jnp.float32).at[:10, :].set(wa[:, :10].T)
    waf = wa[:, 10:].T
    bg1r = bg1[None, :]
    wbe = jnp.zeros((16, 128), jnp.float32).at[:10, :64].set(wb[:, :10].T)
    wbf = jnp.zeros((64, 128), jnp.float32).at[:, :64].set(wb[:, 10:].T)
    wg2t = (Wg2 * (gg2 * s)[:, None]).T
    dup = jnp.asarray(np.tile(np.eye(64, dtype=np.float32), (1, 2)))
    wgpk = jax.scipy.linalg.block_diag(wg2t, wg2t)
    bg2r = bg2[None, :]
    w2f = W2 * (g2 * s)[:, None]
    wsf = Ws * (gs * s)[:, None]
    bsum = (b2 + bs)[None, :]

    nbp = _sc_gather_pack(xyzt, idx_flat, ep, 16)

    grid = npad // _NT
    wspec = lambda shape: pl.BlockSpec(shape, lambda i: (0, 0))
    a_arr, bv_arr, enc_arr = pl.pallas_call(
        _tc1_body,
        grid=(grid,),
        in_specs=[
            pl.BlockSpec((128, _NT), lambda i: (0, i)),
            pl.BlockSpec((_NT, 16), lambda i: (i, 0)),
            pl.BlockSpec((_NT * 2, 128), lambda i: (i, 0)),
            wspec((16, 128)), wspec((128, 128)), wspec((1, 128)),
            wspec((128, 128)), wspec((128, 128)), wspec((128, 128)),
            wspec((128, 128)),
            wspec((128, 64)), wspec((1, 64)),
            wspec((16, 16)), wspec((1, 16)),
            wspec((16, 64)), wspec((64, 64)), wspec((1, 64)), wspec((16, 128)),
            wspec((64, 128)),
        ],
        out_specs=[
            pl.BlockSpec((_NT, 64), lambda i: (i, 0)),
            pl.BlockSpec((_NT, 128), lambda i: (i, 0)),
            pl.BlockSpec((_NT, 16), lambda i: (i, 0)),
        ],
        out_shape=[
            jax.ShapeDtypeStruct((npad, 64), jnp.float32),
            jax.ShapeDtypeStruct((npad, 128), jnp.float32),
            jax.ShapeDtypeStruct((npad, 16), jnp.float32),
        ],
    )(featp, xyzp, nbp, t8, sd, m0c, sha, shb, wfcbd, rmat, w1p, b1r,
      wattp, battp, wae, waf, bg1r, wbe, wbf)

    gbp = _sc_gather_pack(bv_arr, idx_flat, ep, 64)

    out_pm = pl.pallas_call(
        _tc2_body,
        grid=(grid,),
        in_specs=[
            pl.BlockSpec((_NT * 8, 128), lambda i: (i, 0)),
            pl.BlockSpec((_NT, 64), lambda i: (i, 0)),
            pl.BlockSpec((128, _NT), lambda i: (0, i)),
            wspec((64, 128)), wspec((128, 256)), wspec((1, 128)),
            wspec((256, 128)), wspec((256, 128)), wspec((1, 256)),
        ],
        out_specs=pl.BlockSpec((_NT, 256), lambda i: (i, 0)),
        out_shape=jax.ShapeDtypeStruct((npad, 256), jnp.float32),
    )(gbp, a_arr, featp, dup, wgpk, bg2r, w2f, wsf, bsum)

    out = out_pm[:N].T[None, :, :, None]
    enc_out = enc_arr[:N, :10].T[None, :, :, None]
    return out, enc_out


# speedup vs baseline: 1.0958x; 1.0958x over previous
import functools

import jax
import jax.numpy as jnp
import numpy as np
from jax import lax
from jax.experimental import pallas as pl
from jax.experimental.pallas import tpu as pltpu
from jax.experimental.pallas import tpu_sc as plsc

_EPS = 1e-5
_K = 16
_NT = 256
_SPLIT0 = 50


def _sc_gather_pack(table, idx_flat, n_rows, keep):
    info = plsc.get_sparse_core_info()
    nc, ns = info.num_cores, info.num_subcores
    total_chunks = n_rows // 128
    ch0 = ((total_chunks * _SPLIT0) // (ns * 100) // 8) * 8
    ch1 = total_chunks // ns - ch0
    max_ch = max(ch0, ch1, 2)
    pack = 128 // keep
    rows_pc = 128 // pack
    out_rows = n_rows // pack
    idx2 = idx_flat.reshape(total_chunks, 128)
    mesh = plsc.VectorSubcoreMesh(core_axis_name="c", subcore_axis_name="s")

    @functools.partial(
        pl.kernel,
        mesh=mesh,
        out_type=jax.ShapeDtypeStruct((out_rows, 128), jnp.float32),
        scratch_types=[
            pltpu.VMEM((max_ch, 128), jnp.int32),
            pltpu.VMEM((4, 128, 128), jnp.float32),
            pltpu.VMEM((2, rows_pc, 128), jnp.float32),
            pltpu.SemaphoreType.DMA,
            pltpu.SemaphoreType.DMA,
            pltpu.SemaphoreType.DMA,
            pltpu.SemaphoreType.DMA,
            pltpu.SemaphoreType.DMA,
            pltpu.SemaphoreType.DMA,
        ],
    )
    def gk(table_hbm, idx_hbm, out_hbm, idx_v, gbuf, cpk,
           gs0, gs1, gs2, gs3, ss0, ss1):
        cid = lax.axis_index("c")
        sid = lax.axis_index("s")
        gsems = (gs0, gs1, gs2, gs3)
        ssems = (ss0, ss1)

        def compact(b, p):
            def crow(r, carry):
                for l in range(pack):
                    for c4 in range(keep // 16):
                        v = gbuf[b, pack * r + l, pl.ds(c4 * 16, 16)]
                        cpk[p, r, pl.ds(l * keep + c4 * 16, 16)] = v
                return carry
            lax.fori_loop(0, rows_pc, crow, 0)

        def run(cb, cpw):
            if cpw == 0:
                return
            ro = cb * rows_pc
            pltpu.sync_copy(idx_hbm.at[pl.ds(cb, cpw)],
                            idx_v.at[pl.ds(0, cpw)])
            for b in range(4):
                pltpu.async_copy(table_hbm.at[idx_v.at[b]], gbuf.at[b],
                                 gsems[b])

            def step(s, carry):
                for b in range(4):
                    c = 4 * s + b
                    p = b % 2
                    pltpu.make_async_copy(
                        table_hbm.at[idx_v.at[0]], gbuf.at[b], gsems[b]).wait()

                    @pl.when(c >= 2)
                    def _drain_store():
                        pltpu.make_async_copy(
                            cpk.at[p], out_hbm.at[pl.ds(ro, rows_pc)],
                            ssems[p]).wait()

                    compact(b, p)
                    pltpu.async_copy(
                        cpk.at[p],
                        out_hbm.at[pl.ds(ro + c * rows_pc, rows_pc)],
                        ssems[p])

                    @pl.when(c + 4 < cpw)
                    def _next():
                        pltpu.async_copy(
                            table_hbm.at[idx_v.at[c + 4]], gbuf.at[b],
                            gsems[b])
                return carry

            lax.fori_loop(0, cpw // 4, step, 0)
            for p in range(2):
                pltpu.make_async_copy(
                    cpk.at[p], out_hbm.at[pl.ds(ro, rows_pc)], ssems[p]).wait()

        @pl.when(cid == 0)
        def _c0():
            run(sid * ch0, ch0)

        @pl.when(cid == 1)
        def _c1():
            run(ns * ch0 + sid * ch1, ch1)

    return gk(table, idx2)


def _tc1_body(feat, xyzp, nbp_ref, t8, sd, m0, sha, shb, wfcbd, rmat,
              w1p, b1r, wattp, battp, wae, waf, bg1r, wbe, wbf,
              a_ref, bv_ref, enc_ref):
    nt = _NT
    xyz16 = xyzp[...]
    nbp = nbp_ref[...]
    cen1 = jnp.dot(xyz16, t8[...])
    cenp = jnp.broadcast_to(cen1[:, None, :], (nt, 2, 128)).reshape(nt * 2, 128)
    relp = cenp - nbp
    ssq = jnp.dot(relp * relp, sd[...])
    distp = jnp.sqrt(ssq + 1e-12)
    fx = distp * m0[...] + jnp.dot(cenp, sha[...]) + jnp.dot(nbp, shb[...])
    logits = jnp.dot(fx, wfcbd[...])
    e = jnp.exp(logits)
    s2 = jnp.sum(e.reshape(nt, 2, 128), axis=1)
    den = jnp.dot(s2, rmat[...])
    attn = e / jnp.broadcast_to(den[:, None, :], (nt, 2, 128)).reshape(nt * 2, 128)
    w = fx * attn
    fagg_rep = jnp.dot(jnp.sum(w.reshape(nt, 2, 128), axis=1), rmat[...])
    fagg = fagg_rep[:, :16]
    enc = jnp.maximum(jnp.dot(fagg, wattp[...]) + battp[...], 0.0)
    f_pc = jnp.maximum(
        lax.dot_general(feat[...], w1p[...], (((0,), (0,)), ((), ()))) + b1r[...],
        0.0)
    a_ref[...] = jnp.dot(enc, wae[...]) + jnp.dot(f_pc, waf[...]) + bg1r[...]
    bv_ref[...] = jnp.dot(enc, wbe[...]) + jnp.dot(f_pc, wbf[...])
    enc_ref[...] = enc


def _tc2_body(g_ref, a_ref, feat_ref, dup, wgpk, bg2r, w2, ws, bsum, out_ref):
    nt = _NT
    gp = g_ref[...]
    a = a_ref[...]
    aw = jnp.dot(a, dup[...])
    ap = jnp.broadcast_to(aw[:, None, :], (nt, 8, 128)).reshape(nt * 8, 128)
    h = jnp.maximum(gp + ap, 0.0)
    h2 = lax.dot_general(h, wgpk[...], (((1,), (0,)), ((), ())))
    m8 = jnp.max(h2.reshape(nt, 8, 256), axis=1)
    mx = jnp.maximum(jnp.maximum(m8[:, :128], m8[:, 128:]) + bg2r[...], 0.0)
    y = (lax.dot_general(mx, w2[...], (((1,), (1,)), ((), ())))
         + lax.dot_general(feat_ref[...], ws[...], (((0,), (1,)), ((), ())))
         + bsum[...])
    out_ref[...] = jnp.maximum(y, 0.2 * y)


def kernel(feature, xyz, neigh_idx, encode_list, W1, g1, b1, Wfc, Watt, gatt,
           batt, Wg1, gg1, bg1, Wg2, gg2, bg2, W2, g2, b2, Ws, gs, bs):
    del encode_list
    B, d_in, N, _ = feature.shape
    k = neigh_idx.shape[-1]
    npad = ((N + _NT - 1) // _NT) * _NT
    ep = npad * k
    s = 1.0 / np.sqrt(1.0 + _EPS)

    feat = feature[0, :, :, 0]
    featp = jnp.pad(feat, ((0, 0), (0, npad - N)))
    xyzt = jnp.pad(xyz[0], ((0, npad - N), (0, 125)))
    xyzp = jnp.pad(xyz[0], ((0, npad - N), (0, 13)))
    idx_flat = jnp.pad(neigh_idx[0], ((0, npad - N), (0, 0))).astype(
        jnp.int32).reshape(ep)

    w1p = (W1 * (g1 * s)[:, None]).T
    b1r = b1[None, :]
    m0_np = np.zeros((1, 128), np.float32)
    m0_np[0, 0::16] = 1.0
    rmat_np = np.zeros((128, 128), np.float32)
    t8_np = np.zeros((16, 128), np.float32)
    sd_np = np.zeros((128, 128), np.float32)
    sh1_np = np.zeros((128, 128), np.float32)
    sh4_np = np.zeros((128, 128), np.float32)
    sh7_np = np.zeros((128, 128), np.float32)
    for kb in range(8):
        o = 16 * kb
        for c in range(16):
            t8_np[c, o + c] = 1.0
        for c in range(3):
            sd_np[o + c, o] = 1.0
            sh1_np[o + c, o + 1 + c] = 1.0
            sh4_np[o + c, o + 4 + c] = 1.0
            sh7_np[o + c, o + 7 + c] = 1.0
    for i in range(128):
        for j in range(i % 16, 128, 16):
            rmat_np[i, j] = 1.0
    m0c = jnp.asarray(m0_np)
    rmat = jnp.asarray(rmat_np)
    t8 = jnp.asarray(t8_np)
    sd = jnp.asarray(sd_np)
    sha = jnp.asarray(sh1_np + sh4_np)
    shb = jnp.asarray(sh7_np - sh1_np)
    wfcp = jnp.zeros((16, 16), jnp.float32).at[:10, :10].set(Wfc.T)
    wfcbd = jax.scipy.linalg.block_diag(*([wfcp] * 8))
    wattf = Watt * (gatt * s)[:, None]
    wattp = jnp.zeros((16, 16), jnp.float32).at[:10, :10].set(wattf.T)
    battp = jnp.zeros((1, 16), jnp.float32).at[0, :10].set(batt)
    wg1f = Wg1 * (gg1 * s)[:, None]
    wa = wg1f[:, :74] - wg1f[:, 74:]
    wb = wg1f[:, 74:]
    wae = jnp.zeros((16, 64), jnp.float32).at[:10, :].set(wa[:, :10].T)
    waf = wa[:, 10:].T
    bg1r = bg1[None, :]
    wbe = jnp.zeros((16, 128), jnp.float32).at[:10, :64].set(wb[:, :10].T)
    wbf = jnp.zeros((64, 128), jnp.float32).at[:, :64].set(wb[:, 10:].T)
    wg2t = (Wg2 * (gg2 * s)[:, None]).T
    dup = jnp.asarray(np.tile(np.eye(64, dtype=np.float32), (1, 2)))
    wgpk = jax.scipy.linalg.block_diag(wg2t, wg2t)
    bg2r = bg2[None, :]
    w2f = W2 * (g2 * s)[:, None]
    wsf = Ws * (gs * s)[:, None]
    bsum = (b2 + bs)[None, :]

    nbp = _sc_gather_pack(xyzt, idx_flat, ep, 16)

    grid = npad // _NT
    wspec = lambda shape: pl.BlockSpec(shape, lambda i: (0, 0))
    a_arr, bv_arr, enc_arr = pl.pallas_call(
        _tc1_body,
        grid=(grid,),
        in_specs=[
            pl.BlockSpec((128, _NT), lambda i: (0, i)),
            pl.BlockSpec((_NT, 16), lambda i: (i, 0)),
            pl.BlockSpec((_NT * 2, 128), lambda i: (i, 0)),
            wspec((16, 128)), wspec((128, 128)), wspec((1, 128)),
            wspec((128, 128)), wspec((128, 128)), wspec((128, 128)),
            wspec((128, 128)),
            wspec((128, 64)), wspec((1, 64)),
            wspec((16, 16)), wspec((1, 16)),
            wspec((16, 64)), wspec((64, 64)), wspec((1, 64)), wspec((16, 128)),
            wspec((64, 128)),
        ],
        out_specs=[
            pl.BlockSpec((_NT, 64), lambda i: (i, 0)),
            pl.BlockSpec((_NT, 128), lambda i: (i, 0)),
            pl.BlockSpec((_NT, 16), lambda i: (i, 0)),
        ],
        out_shape=[
            jax.ShapeDtypeStruct((npad, 64), jnp.float32),
            jax.ShapeDtypeStruct((npad, 128), jnp.float32),
            jax.ShapeDtypeStruct((npad, 16), jnp.float32),
        ],
    )(featp, xyzp, nbp, t8, sd, m0c, sha, shb, wfcbd, rmat, w1p, b1r,
      wattp, battp, wae, waf, bg1r, wbe, wbf)

    gbp = _sc_gather_pack(bv_arr, idx_flat, ep, 64)

    out_pm = pl.pallas_call(
        _tc2_body,
        grid=(grid,),
        in_specs=[
            pl.BlockSpec((_NT * 8, 128), lambda i: (i, 0)),
            pl.BlockSpec((_NT, 64), lambda i: (i, 0)),
            pl.BlockSpec((128, _NT), lambda i: (0, i)),
            wspec((64, 128)), wspec((128, 256)), wspec((1, 128)),
            wspec((256, 128)), wspec((256, 128)), wspec((1, 256)),
        ],
        out_specs=pl.BlockSpec((_NT, 256), lambda i: (i, 0)),
        out_shape=jax.ShapeDtypeStruct((npad, 256), jnp.float32),
    )(gbp, a_arr, featp, dup, wgpk, bg2r, w2f, wsf, bsum)

    out = out_pm[:N].T[None, :, :, None]
    enc_out = enc_arr[:N, :10].T[None, :, :, None]
    return out, enc_out
